# baseline (speedup 1.0000x reference)
import functools

import jax
import jax.numpy as jnp
from jax import lax
from jax.experimental import pallas as pl
from jax.experimental.pallas import tpu as pltpu
from jax.experimental.pallas import tpu_sc as plsc

N = 10000
E = 320000
NF = 128
OC = 128
LD = 256
T_IN = 12
T_OUT = 12
TP = T_IN // 2
NF2 = 2 * NF

NSC = 2
NTILE = 16
TILE_E = E // NTILE
SB = 32
NBUF = 3
BATCH = NBUF * SB
EBUF = TILE_E + 2 * BATCH
NPADR = 10240
RSLICE = NPADR // NTILE
NQUART = 2504
NP = 4 * NQUART
ACCR = 2560
ZROWS = ACCR // NTILE
WSL = 160
WTAIL = NQUART - (NTILE - 1) * WSL
DSHIFT = 14
DMASK = (1 << DSHIFT) - 1

_BLK = 400


def _rsqrt_newton(v):
    xi = lax.bitcast_convert_type(v, jnp.int32)
    y = lax.bitcast_convert_type(jnp.int32(0x5F3759DF) - (xi >> 1), jnp.float32)
    for _ in range(3):
        y = y * (1.5 - 0.5 * v * y * y)
    return y


def _sc_body(xpair_h, pk_h, src_h, dst_h, ew_h, ones_h,
             ax_h, dv2_h, nrm_h,
             pk_v, nrm_v, dinv_v, slice_v, d2_v,
             rows0, rows1, rows2,
             outsA0, outsA1, outsA2, outsB0, outsB1, outsB2,
             gidx_v, didx_v,
             deg_s, accA_s, accB_s,
             gsem0, gsem1, gsem2, ssem0, ssem1, ssem2):
    c = lax.axis_index("c")
    s = lax.axis_index("s")
    ebase = s * TILE_E
    rbase = s * RSLICE

    pltpu.sync_copy(dst_h.at[pl.ds(ebase, TILE_E)], pk_v.at[pl.ds(0, TILE_E)])
    pltpu.sync_copy(ew_h.at[pl.ds(ebase, TILE_E)], nrm_v.at[pl.ds(0, TILE_E)])

    gtail = jnp.full((16,), N, jnp.int32)
    ztail = jnp.zeros((16,), jnp.float32)
    for j in range((EBUF - TILE_E) // 16):
        tsl = pl.ds(TILE_E + j * 16, 16)
        pk_v[tsl] = gtail
        nrm_v[tsl] = ztail

    pltpu.sync_copy(ones_h.at[pl.ds(0, RSLICE)], slice_v)
    pltpu.sync_copy(slice_v, deg_s.at[pl.ds(rbase, RSLICE)])
    plsc.subcore_barrier()
    pltpu.sync_copy(nrm_v, deg_s.at[pk_v], add=True)
    plsc.subcore_barrier()

    pltpu.sync_copy(deg_s.at[pl.ds(rbase, RSLICE)], slice_v)

    def dinv_body(j, _):
        sl = pl.ds(j * 16, 16)
        y = _rsqrt_newton(slice_v[sl])
        slice_v[sl] = y
        d2_v[sl] = y * y
        return 0

    lax.fori_loop(0, RSLICE // 16, dinv_body, 0)
    pltpu.sync_copy(slice_v, deg_s.at[pl.ds(rbase, RSLICE)])

    @pl.when(c == 0)
    def _():
        pltpu.sync_copy(d2_v, dv2_h.at[pl.ds(rbase, RSLICE)])

    plsc.subcore_barrier()

    pltpu.sync_copy(deg_s.at[pl.ds(0, N)], dinv_v)

    def dfac_body(j, _):
        sl = pl.ds(j * 16, 16)
        b = plsc.load_gather(dinv_v, [pk_v[sl]])
        nrm_v[sl] = nrm_v[sl] * b
        return 0

    lax.fori_loop(0, TILE_E // 16, dfac_body, 0)
    pltpu.sync_copy(src_h.at[pl.ds(ebase, TILE_E)], pk_v.at[pl.ds(0, TILE_E)])

    def sfac_body(j, _):
        sl = pl.ds(j * 16, 16)
        a = plsc.load_gather(dinv_v, [pk_v[sl]])
        nrm_v[sl] = nrm_v[sl] * a
        return 0

    lax.fori_loop(0, TILE_E // 16, sfac_body, 0)
    pltpu.sync_copy(nrm_v.at[pl.ds(0, TILE_E)], nrm_h.at[pl.ds(ebase, TILE_E)])

    rows = [rows0, rows1, rows2]
    outsA = [outsA0, outsA1, outsA2]
    outsB = [outsB0, outsB1, outsB2]
    gsem = [gsem0, gsem1, gsem2]
    ssem = [ssem0, ssem1, ssem2]

    def quarter_body(g, _):
        qlo = (2 * c + g) * NQUART

        pltpu.sync_copy(pk_h.at[pl.ds(ebase, TILE_E)],
                        pk_v.at[pl.ds(0, TILE_E)])
        pltpu.sync_copy(nrm_h.at[pl.ds(ebase, TILE_E)],
                        nrm_v.at[pl.ds(0, TILE_E)])

        def part_body(j, carry):
            sl = pl.ds(j * 16, 16)
            w16 = pk_v[sl]
            n16 = nrm_v[sl]
            d16 = (w16 & DMASK) - qlo
            m = (d16 >= 0) & (d16 < NQUART)
            wsl = pl.ds(carry, 16)
            plsc.store_compressed(pk_v.at[wsl], w16, mask=m)
            plsc.store_compressed(nrm_v.at[wsl], n16, mask=m)
            return carry + jnp.sum(m.astype(jnp.int32))

        cnt = lax.fori_loop(0, TILE_E // 16, part_body, 0)
        ng = jnp.maximum((cnt + BATCH - 1) // BATCH, 1)
        gz = jnp.zeros((16,), jnp.float32)
        gpad = ((qlo + NQUART + lax.iota(jnp.int32, 16)) &
                jnp.int32(DMASK))
        for j in range(BATCH // 16):
            psl = pl.ds(cnt + j * 16, 16)
            pk_v[psl] = gpad
            nrm_v[psl] = gz

        tz = jnp.minimum(cnt, 0)

        def ioff(jb):
            return (jnp.int32(jb) + tz) * SB

        def prep(m, jb, pN):
            for i in range(SB // 16):
                w16 = pk_v[pl.ds(m * SB + i * 16, 16)]
                gidx_v[pl.ds(jb * SB + i * 16, 16)] = (w16 >> DSHIFT) + pN
                didx_v[pl.ds(jb * SB + i * 16, 16)] = (w16 & DMASK) - qlo

        def gather_start(jb):
            pltpu.async_copy(xpair_h.at[gidx_v.at[pl.ds(ioff(jb), SB)]],
                             rows[jb], gsem[jb])

        def gather_wait(jb):
            pltpu.make_async_copy(xpair_h.at[gidx_v.at[pl.ds(ioff(jb), SB)]],
                                  rows[jb], gsem[jb]).wait()

        def scatter_start(jb):
            pltpu.async_copy(outsA[jb],
                             accA_s.at[didx_v.at[pl.ds(ioff(jb), SB)]],
                             ssem[jb], add=True)
            pltpu.async_copy(outsB[jb],
                             accB_s.at[didx_v.at[pl.ds(ioff(jb), SB)]],
                             ssem[jb], add=True)

        def scatter_wait(jb):
            pltpu.make_async_copy(outsA[jb],
                                  accA_s.at[didx_v.at[pl.ds(ioff(jb), SB)]],
                                  ssem[jb]).wait()
            pltpu.make_async_copy(outsB[jb],
                                  accB_s.at[didx_v.at[pl.ds(ioff(jb), SB)]],
                                  ssem[jb]).wait()

        def pair_body(p, _):
            pN = p * N

            z16 = jnp.zeros((16,), jnp.float32)

            def zrow_body(r, _):
                for v in range(NF // 16):
                    outsA0[r, pl.ds(v * 16, 16)] = z16
                return 0

            lax.fori_loop(0, SB, zrow_body, 0)
            abase = s * ZROWS
            for r in range(ZROWS // SB):
                pltpu.sync_copy(outsA0, accA_s.at[pl.ds(abase + r * SB, SB)])
                pltpu.sync_copy(outsA0, accB_s.at[pl.ds(abase + r * SB, SB)])
            plsc.subcore_barrier()

            prep(0, 0, pN)
            gather_start(0)
            prep(1, 1, pN)
            gather_start(1)

            def group_body(q, _):
                for j in range(NBUF):
                    k = NBUF * q + j
                    gather_wait(j)
                    eb = k * SB

                    def scale_body(j2, _):
                        for u in range(4):
                            e = j2 * 4 + u
                            nv = plsc.load_gather(
                                nrm_v, [jnp.full((16,), eb + e, jnp.int32)])
                            for v in range(NF // 16):
                                sl = pl.ds(v * 16, 16)
                                w = rows[j][e, sl]
                                bfv = plsc.bitcast(w, jnp.bfloat16)
                                a, b = plsc.unpack(
                                    bfv, format=plsc.PackFormat.INTERLEAVED)
                                outsA[j][e, sl] = a * nv
                                outsB[j][e, sl] = b * nv
                        return 0

                    lax.fori_loop(0, SB // 4, scale_body, 0)
                    scatter_start(j)
                    jb = (j + 2) % NBUF
                    if j == 0:
                        @pl.when(q >= 1)
                        def _():
                            scatter_wait(jb)
                        prep(k + 2, jb, pN)
                        gather_start(jb)
                    else:
                        @pl.when(q + 1 < ng)
                        def _():
                            scatter_wait(jb)
                            prep(k + 2, jb, pN)
                            gather_start(jb)
                return 0

            lax.fori_loop(0, ng, group_body, 0)
            for j in range(NBUF):
                scatter_wait(j)
            plsc.subcore_barrier()

            ooffa = 2 * p * NP + qlo
            ooffb = (2 * p + 1) * NP + qlo
            wbase = s * WSL

            @pl.when(s < NTILE - 1)
            def _():
                pltpu.sync_copy(accA_s.at[pl.ds(wbase, WSL)],
                                ax_h.at[pl.ds(ooffa + wbase, WSL)])
                pltpu.sync_copy(accB_s.at[pl.ds(wbase, WSL)],
                                ax_h.at[pl.ds(ooffb + wbase, WSL)])

            @pl.when(s == NTILE - 1)
            def _():
                tb = (NTILE - 1) * WSL
                pltpu.sync_copy(accA_s.at[pl.ds(tb, WTAIL)],
                                ax_h.at[pl.ds(ooffa + tb, WTAIL)])
                pltpu.sync_copy(accB_s.at[pl.ds(tb, WTAIL)],
                                ax_h.at[pl.ds(ooffb + tb, WTAIL)])

            plsc.subcore_barrier()
            return 0

        lax.fori_loop(0, TP, pair_body, 0)
        return 0

    lax.fori_loop(0, 2, quarter_body, 0)


def _spmm_sc(xpair, pk, src, dst, ew, ones_a):
    mesh = plsc.VectorSubcoreMesh(core_axis_name="c", subcore_axis_name="s")
    f = pl.kernel(
        _sc_body,
        out_type=[
            jax.ShapeDtypeStruct((T_IN * NP, NF), jnp.float32),
            jax.ShapeDtypeStruct((NPADR,), jnp.float32),
            jax.ShapeDtypeStruct((E,), jnp.float32),
        ],
        mesh=mesh,
        compiler_params=pltpu.CompilerParams(needs_layout_passes=False),
        scratch_types=[
            pltpu.VMEM((EBUF,), jnp.int32),
            pltpu.VMEM((EBUF,), jnp.float32),
            pltpu.VMEM((N,), jnp.float32),
            pltpu.VMEM((RSLICE,), jnp.float32),
            pltpu.VMEM((RSLICE,), jnp.float32),
            pltpu.VMEM((SB, NF), jnp.int32),
            pltpu.VMEM((SB, NF), jnp.int32),
            pltpu.VMEM((SB, NF), jnp.int32),
            pltpu.VMEM((SB, NF), jnp.float32),
            pltpu.VMEM((SB, NF), jnp.float32),
            pltpu.VMEM((SB, NF), jnp.float32),
            pltpu.VMEM((SB, NF), jnp.float32),
            pltpu.VMEM((SB, NF), jnp.float32),
            pltpu.VMEM((SB, NF), jnp.float32),
            pltpu.VMEM((NBUF * SB,), jnp.int32),
            pltpu.VMEM((NBUF * SB,), jnp.int32),
            pltpu.VMEM_SHARED((NPADR,), jnp.float32),
            pltpu.VMEM_SHARED((ACCR, NF), jnp.float32),
            pltpu.VMEM_SHARED((ACCR, NF), jnp.float32),
            pltpu.SemaphoreType.DMA,
            pltpu.SemaphoreType.DMA,
            pltpu.SemaphoreType.DMA,
            pltpu.SemaphoreType.DMA,
            pltpu.SemaphoreType.DMA,
            pltpu.SemaphoreType.DMA,
        ],
    )
    return f(xpair, pk, src, dst, ew, ones_a)


def _gru_head_body(ax_ref, xt_ref, dv2_ref,
                   Wz_ref, bz_ref, Wr_ref, br_ref, Wh_ref, bh_ref,
                   lzW_ref, lzb_ref, lrW_ref, lrb_ref, lhW_ref, lhb_ref,
                   l1W_ref, l1b_ref, l2W_ref, l2b_ref, out_ref):
    f32 = jnp.float32
    dot = functools.partial(jnp.dot, preferred_element_type=f32)
    dv2 = dv2_ref[:]
    H = jnp.zeros((_BLK, OC), f32)
    for t in range(T_IN):
        C = ax_ref[t] + dv2 * xt_ref[t]
        Gz = dot(C, Wz_ref[:]) + bz_ref[:]
        Gr = dot(C, Wr_ref[:]) + br_ref[:]
        Gh = dot(C, Wh_ref[:]) + bh_ref[:]
        Z = jax.nn.sigmoid(dot(Gz, lzW_ref[:OC]) + dot(H, lzW_ref[OC:]) + lzb_ref[:])
        R = jax.nn.sigmoid(dot(Gr, lrW_ref[:OC]) + dot(H, lrW_ref[OC:]) + lrb_ref[:])
        Ht = jnp.tanh(dot(Gh, lhW_ref[:OC]) + dot(H * R, lhW_ref[OC:]) + lhb_ref[:])
        H = Z * H + (1.0 - Z) * Ht
    h = jax.nn.relu(H)
    h = jax.nn.relu(dot(h, l1W_ref[:]) + l1b_ref[:])
    out_ref[:] = dot(h, l2W_ref[:]) + l2b_ref[:]


def _gru_head(ax, xt, dv2, Wz, bz, Wr, br, Wh, bh,
              lzW, lzb, lrW, lrb, lhW, lhb, l1W, l1b, l2W, l2b):
    grid = N // _BLK
    full = lambda shape: pl.BlockSpec(shape, lambda i: (0,) * len(shape))
    return pl.pallas_call(
        _gru_head_body,
        grid=(grid,),
        in_specs=[
            pl.BlockSpec((T_IN, _BLK, NF), lambda i: (0, i, 0)),
            pl.BlockSpec((T_IN, _BLK, NF), lambda i: (0, i, 0)),
            pl.BlockSpec((_BLK, 1), lambda i: (i, 0)),
            full((NF, OC)), full((1, OC)),
            full((NF, OC)), full((1, OC)),
            full((NF, OC)), full((1, OC)),
            full((2 * OC, OC)), full((1, OC)),
            full((2 * OC, OC)), full((1, OC)),
            full((2 * OC, OC)), full((1, OC)),
            full((OC, LD)), full((1, LD)),
            full((LD, T_OUT)), full((1, T_OUT)),
        ],
        out_specs=pl.BlockSpec((_BLK, T_OUT), lambda i: (i, 0)),
        out_shape=jax.ShapeDtypeStruct((N, T_OUT), jnp.float32),
        compiler_params=pltpu.CompilerParams(
            dimension_semantics=("arbitrary",),
        ),
    )(ax, xt, dv2, Wz, bz, Wr, br, Wh, bh,
      lzW, lzb, lrW, lrb, lhW, lhb, l1W, l1b, l2W, l2b)


def kernel(x, edge_index, edge_attr, W_z, b_z, W_r, b_r, W_h, b_h,
           lz_W, lz_b, lr_W, lr_b, lh_W, lh_b, l1_W, l1_b, l2_W, l2_b):
    src = edge_index[0].astype(jnp.int32)
    dst = edge_index[1].astype(jnp.int32)
    ew = edge_attr
    pk = (src << DSHIFT) | dst

    xT = jnp.transpose(x, (2, 0, 1))
    xb = x.astype(jnp.bfloat16)
    xpz = jnp.stack([xb[:, :, 0::2], xb[:, :, 1::2]], axis=-1)
    xi = lax.bitcast_convert_type(xpz, jnp.int32)
    xpair = jnp.transpose(xi, (2, 0, 1)).reshape(TP * N, NF)
    ones_a = jnp.ones((RSLICE,), jnp.float32)

    axflat, dv2, _ = _spmm_sc(xpair, pk, src, dst, ew, ones_a)
    AX = axflat.reshape(T_IN, NP, NF)[:, :N, :]
    dv2 = dv2[:N].reshape(N, 1)

    r2 = lambda v: v.reshape(1, -1)
    return _gru_head(AX, xT, dv2,
                     W_z, r2(b_z), W_r, r2(b_r), W_h, r2(b_h),
                     lz_W, r2(lz_b), lr_W, r2(lr_b), lh_W, r2(lh_b),
                     l1_W, r2(l1_b), l2_W, r2(l2_b))
